# baseline (speedup 1.0000x reference)
import functools

import jax
import jax.numpy as jnp
from jax import lax
from jax.experimental import pallas as pl
from jax.experimental.pallas import tpu as pltpu
from jax.experimental.pallas import tpu_sc as plsc

B, S, H = 4, 4096, 2048
NC, NS = 2, 16
NW = NC * NS
POS_PER_W = S // NW
CH = 8
NCHUNK = POS_PER_W // CH
LANES = 16
INNER = H // LANES


def _sc_body(x_hbm, pe_hbm, out_hbm, pe_v, x_v):
    wid = lax.axis_index("s") * NC + lax.axis_index("c")
    base = wid * POS_PER_W

    def chunk_loop(k, _):
        start = base + k * CH
        pltpu.sync_copy(pe_hbm.at[pl.ds(start, CH), :], pe_v)
        for b in range(B):
            pltpu.sync_copy(x_hbm.at[b, pl.ds(start, CH), :], x_v)
            for r in range(CH):
                def add_loop(j, _):
                    sl = pl.ds(j * LANES, LANES)
                    x_v[r, sl] = x_v[r, sl] + pe_v[r, sl]
                    return 0
                lax.fori_loop(0, INNER, add_loop, 0)
            pltpu.sync_copy(x_v, out_hbm.at[b, pl.ds(start, CH), :])
        return 0

    lax.fori_loop(0, NCHUNK, chunk_loop, 0)


_sc_call = functools.partial(
    pl.kernel,
    out_type=jax.ShapeDtypeStruct((B, S, H), jnp.float32),
    mesh=plsc.VectorSubcoreMesh(core_axis_name="c", subcore_axis_name="s"),
    scratch_types=[
        pltpu.VMEM((CH, H), jnp.float32),
        pltpu.VMEM((CH, H), jnp.float32),
    ],
)(_sc_body)


def kernel(inputs, pe):
    return _sc_call(inputs, pe)


# speedup vs baseline: 1.1744x; 1.1744x over previous
import functools

import jax
import jax.numpy as jnp
from jax import lax
from jax.experimental import pallas as pl
from jax.experimental.pallas import tpu as pltpu
from jax.experimental.pallas import tpu_sc as plsc

B, S, H = 4, 4096, 2048
MAX_POS = 8192
NC, NS = 2, 16
NW = NC * NS
POS_PER_W = S // NW
CH = 4
NCHUNK = POS_PER_W // CH
CHH = CH * H
LANES = 16
VECS = CHH // LANES


def _sc_body(x_hbm, pe_hbm, out_hbm, *scratch):
    xb = [[scratch[2 * b + p] for p in range(2)] for b in range(4)]
    pe_v = [scratch[8], scratch[9]]
    in_sem = [[scratch[10 + 2 * b + p] for p in range(2)] for b in range(4)]
    out_sem = [[scratch[18 + 2 * b + p] for p in range(2)] for b in range(4)]
    pe_sem = [scratch[26], scratch[27]]

    wid = lax.axis_index("s") * NC + lax.axis_index("c")
    base = wid * POS_PER_W

    def pe_off(k):
        return pl.multiple_of((base + k * CH) * H, 8)

    def x_off(b, k):
        return pl.multiple_of(b * (S * H) + (base + k * CH) * H, 8)

    pltpu.async_copy(pe_hbm.at[pl.ds(pe_off(0), CHH)], pe_v[0], pe_sem[0])
    for b in range(4):
        pltpu.async_copy(
            x_hbm.at[pl.ds(x_off(b, 0), CHH)], xb[b][0], in_sem[b][0]
        )

    def chunk_pair(k2, _):
        for j in range(2):
            k = 2 * k2 + j
            p = j
            q = 1 - j
            pltpu.make_async_copy(
                pe_hbm.at[pl.ds(pe_off(k), CHH)], pe_v[p], pe_sem[p]
            ).wait()
            if j == 0:
                pltpu.async_copy(
                    pe_hbm.at[pl.ds(pe_off(k + 1), CHH)], pe_v[q], pe_sem[q]
                )
            else:
                @pl.when(k2 < NCHUNK // 2 - 1)
                def _():
                    pltpu.async_copy(
                        pe_hbm.at[pl.ds(pe_off(k + 1), CHH)], pe_v[q], pe_sem[q]
                    )
            for b in range(4):
                pltpu.make_async_copy(
                    x_hbm.at[pl.ds(x_off(b, k), CHH)], xb[b][p], in_sem[b][p]
                ).wait()

                xref = xb[b][p]
                peref = pe_v[p]

                @plsc.parallel_loop(0, VECS, step=1, unroll=8)
                def _(i):
                    sl = pl.ds(i * LANES, LANES)
                    xref[sl] = xref[sl] + peref[sl]

                pltpu.async_copy(
                    xref, out_hbm.at[pl.ds(x_off(b, k), CHH)], out_sem[b][p]
                )
                if j == 0:
                    @pl.when(k2 >= 1)
                    def _():
                        pltpu.make_async_copy(
                            xb[b][q],
                            out_hbm.at[pl.ds(x_off(b, k), CHH)],
                            out_sem[b][q],
                        ).wait()

                    pltpu.async_copy(
                        x_hbm.at[pl.ds(x_off(b, k + 1), CHH)],
                        xb[b][q],
                        in_sem[b][q],
                    )
                else:
                    @pl.when(k2 < NCHUNK // 2 - 1)
                    def _():
                        pltpu.make_async_copy(
                            xb[b][q],
                            out_hbm.at[pl.ds(x_off(b, k), CHH)],
                            out_sem[b][q],
                        ).wait()
                        pltpu.async_copy(
                            x_hbm.at[pl.ds(x_off(b, k + 1), CHH)],
                            xb[b][q],
                            in_sem[b][q],
                        )
        return 0

    lax.fori_loop(0, NCHUNK // 2, chunk_pair, 0)

    for b in range(4):
        pltpu.make_async_copy(
            xb[b][0],
            out_hbm.at[pl.ds(x_off(b, NCHUNK - 2), CHH)],
            out_sem[b][0],
        ).wait()
        pltpu.make_async_copy(
            xb[b][1],
            out_hbm.at[pl.ds(x_off(b, NCHUNK - 1), CHH)],
            out_sem[b][1],
        ).wait()


_vmem = [pltpu.VMEM((CHH,), jnp.float32) for _ in range(10)]
_sems = [pltpu.SemaphoreType.DMA for _ in range(18)]

_sc_call = functools.partial(
    pl.kernel,
    out_type=jax.ShapeDtypeStruct((B * S * H,), jnp.float32),
    mesh=plsc.VectorSubcoreMesh(core_axis_name="c", subcore_axis_name="s"),
    scratch_types=_vmem + _sems,
)(_sc_body)


def kernel(inputs, pe):
    out = _sc_call(inputs.reshape(-1), pe.reshape(-1))
    return out.reshape(B, S, H)


# speedup vs baseline: 3.7992x; 3.2352x over previous
import functools

import jax
import jax.numpy as jnp
from jax import lax
from jax.experimental import pallas as pl
from jax.experimental.pallas import tpu as pltpu
from jax.experimental.pallas import tpu_sc as plsc

B, S, H = 4, 4096, 2048
NC, NS = 2, 16
NW = NC * NS
POS_PER_W = S // NW
CR = 8
CW = H // 2
NPAIR = POS_PER_W // CR
LANES = 16
CVECS = CW // LANES


def _sc_body(x_hbm, pe_hbm, out_hbm, *scratch):
    xb = [[scratch[2 * b + p] for p in range(2)] for b in range(4)]
    pe_v = [scratch[8], scratch[9]]
    in_sem = [[scratch[10 + 2 * b + p] for p in range(2)] for b in range(4)]
    out_sem = [[scratch[18 + 2 * b + p] for p in range(2)] for b in range(4)]
    pe_sem = [scratch[26], scratch[27]]

    wid = lax.axis_index("s") * NC + lax.axis_index("c")
    base = wid * POS_PER_W

    def row0(u2):
        return pl.multiple_of(base + u2 * CR, 8)


    pltpu.async_copy(
        pe_hbm.at[pl.ds(row0(0), CR), pl.ds(0, CW)], pe_v[0], pe_sem[0]
    )
    for b in range(4):
        pltpu.async_copy(
            x_hbm.at[b, pl.ds(row0(0), CR), pl.ds(0, CW)],
            xb[b][0],
            in_sem[b][0],
        )

    def pair_body(u2, _):
        for j in range(2):
            p = j
            q = 1 - j
            r0 = row0(u2)
            c0 = j * CW
            nr0 = r0 if j == 0 else row0(u2 + 1)
            nc0 = CW if j == 0 else 0
            pltpu.make_async_copy(
                pe_hbm.at[pl.ds(r0, CR), pl.ds(c0, CW)], pe_v[p], pe_sem[p]
            ).wait()
            if j == 0:
                pltpu.async_copy(
                    pe_hbm.at[pl.ds(nr0, CR), pl.ds(nc0, CW)],
                    pe_v[q],
                    pe_sem[q],
                )
            else:
                @pl.when(u2 < NPAIR - 1)
                def _():
                    pltpu.async_copy(
                        pe_hbm.at[pl.ds(nr0, CR), pl.ds(nc0, CW)],
                        pe_v[q],
                        pe_sem[q],
                    )
            for b in range(4):
                pltpu.make_async_copy(
                    x_hbm.at[b, pl.ds(r0, CR), pl.ds(c0, CW)],
                    xb[b][p],
                    in_sem[b][p],
                ).wait()

                xref = xb[b][p]
                peref = pe_v[p]

                @plsc.parallel_loop(0, CVECS, step=1, unroll=2)
                def _(i):
                    sl = pl.ds(i * LANES, LANES)
                    for r in range(CR):
                        xref[r, sl] = xref[r, sl] + peref[r, sl]

                pltpu.async_copy(
                    xref,
                    out_hbm.at[b, pl.ds(r0, CR), pl.ds(c0, CW)],
                    out_sem[b][p],
                )
                if j == 0:
                    @pl.when(u2 >= 1)
                    def _():
                        pltpu.make_async_copy(
                            xb[b][q],
                            out_hbm.at[b, pl.ds(r0, CR), pl.ds(c0, CW)],
                            out_sem[b][q],
                        ).wait()

                    pltpu.async_copy(
                        x_hbm.at[b, pl.ds(nr0, CR), pl.ds(nc0, CW)],
                        xb[b][q],
                        in_sem[b][q],
                    )
                else:
                    @pl.when(u2 < NPAIR - 1)
                    def _():
                        pltpu.make_async_copy(
                            xb[b][q],
                            out_hbm.at[b, pl.ds(r0, CR), pl.ds(c0, CW)],
                            out_sem[b][q],
                        ).wait()
                        pltpu.async_copy(
                            x_hbm.at[b, pl.ds(nr0, CR), pl.ds(nc0, CW)],
                            xb[b][q],
                            in_sem[b][q],
                        )
        return 0

    lax.fori_loop(0, NPAIR, pair_body, 0)

    for b in range(4):
        pltpu.make_async_copy(
            xb[b][0],
            out_hbm.at[b, pl.ds(row0(NPAIR - 1), CR), pl.ds(0, CW)],
            out_sem[b][0],
        ).wait()
        pltpu.make_async_copy(
            xb[b][1],
            out_hbm.at[b, pl.ds(row0(NPAIR - 1), CR), pl.ds(CW, CW)],
            out_sem[b][1],
        ).wait()


_vmem = [pltpu.VMEM((CR, CW), jnp.float32) for _ in range(10)]
_sems = [pltpu.SemaphoreType.DMA for _ in range(18)]

_sc_call = functools.partial(
    pl.kernel,
    out_type=jax.ShapeDtypeStruct((B, S, H), jnp.float32),
    mesh=plsc.VectorSubcoreMesh(core_axis_name="c", subcore_axis_name="s"),
    scratch_types=_vmem + _sems,
)(_sc_body)


def kernel(inputs, pe):
    return _sc_call(inputs, pe)
